# baseline (speedup 1.0000x reference)
import functools

import numpy as np
import jax
import jax.numpy as jnp
from jax import lax
from jax.experimental import pallas as pl
from jax.experimental.pallas import tpu as pltpu
from jax.experimental.pallas import tpu_sc as plsc

_WINDOW = 96
_SENT = 120
_FS = 500
_DOWN_FS = 100
_NPATCH = _SENT // 12
_MAXPEAKS = 25
_ROWPAD = 1024
_NLEAD = 12
_LANES = 128
_BBLK = 8
_NC = 2
_NS = 16


def _downsample_weights(L=5000, scale=_DOWN_FS / _FS):
    Lo = int(L * scale)
    src = (np.arange(Lo) + 0.5) / scale - 0.5
    src = np.clip(src, 0.0, L - 1)
    lo = np.floor(src).astype(np.int64)
    hi = np.minimum(lo + 1, L - 1)
    frac = (src - lo).astype(np.float32)
    base = 5 * np.arange(Lo)
    olo = lo - base - 1
    ohi = hi - base - 1
    assert olo.min() >= 0 and ohi.max() <= 2, "static interp offsets changed"
    w = np.zeros((3, Lo), np.float32)
    w[olo, np.arange(Lo)] += (1.0 - frac)
    w[ohi, np.arange(Lo)] += frac
    return w


_W3 = _downsample_weights()
_LD = _W3.shape[1]


def _qrs_body(xt0_ref, xtt_ref, w_ref, wt_ref,
              tbl_ref, idx_ref, t_ref, s_ref):
    B = xt0_ref.shape[0]
    Ld = _LD
    W = _WINDOW
    P = _NPATCH
    S = _SENT

    xt0 = xt0_ref[...]
    w = w_ref[...]
    sig = (xt0[:, 0] * w[0:1, :] + xt0[:, 1] * w[1:2, :]
           + xt0[:, 2] * w[2:3, :])

    xtt = xtt_ref[...]
    wt = wt_ref[...]
    xds_t = (xtt[:, 0] * wt[0] + xtt[:, 1] * wt[1]
             + xtt[:, 2] * wt[2])
    tbl_ref[...] = jnp.zeros((B, _ROWPAD, _LANES), jnp.float32)
    tbl_ref[:, 0:Ld, 0:_NLEAD] = xds_t

    mu = jnp.mean(sig, axis=1, keepdims=True)
    var = jnp.mean((sig - mu) ** 2, axis=1, keepdims=True)
    thr = mu + 1.5 * jnp.sqrt(var)
    sl = jnp.concatenate([sig[:, :1], sig[:, :-1]], axis=1)
    sr = jnp.concatenate([sig[:, 1:], sig[:, -1:]], axis=1)
    ii = jax.lax.broadcasted_iota(jnp.int32, (B, Ld), 1)
    cand = ((sig > sl) & (sig >= sr) & (sig > thr)
            & (ii >= 1) & (ii <= Ld - 2))

    amp = jnp.where(cand, sig, -jnp.inf)
    kept = jnp.zeros((B, Ld), dtype=jnp.bool_)
    for _ in range(_MAXPEAKS):
        rowmax = jnp.max(amp, axis=1, keepdims=True)
        valid = rowmax > -jnp.inf
        pm = jnp.min(jnp.where(amp == rowmax, ii, Ld), axis=1, keepdims=True)
        kept = kept | (valid & (ii == pm))
        amp = jnp.where(valid & (jnp.abs(ii - pm) < 40), -jnp.inf, amp)

    has = jnp.any(kept, axis=1, keepdims=True)
    smax = jnp.max(sig, axis=1, keepdims=True)
    pmax = jnp.min(jnp.where(sig == smax, ii, Ld), axis=1, keepdims=True)
    kept = (kept & has) | ((ii == pmax) & jnp.logical_not(has))
    m = jnp.sum(kept.astype(jnp.int32), axis=1, keepdims=True)

    cols = []
    kk = kept
    for _ in range(_MAXPEAKS):
        ps = jnp.min(jnp.where(kk, ii, Ld), axis=1, keepdims=True)
        cols.append(jnp.where(ps < Ld, ps, 0))
        kk = kk & (ii != ps)
    qrs = jnp.concatenate(cols, axis=1)

    pp = jax.lax.broadcasted_iota(jnp.int32, (B, W), 1)
    idx_cols = []
    for j in range(P):
        qj = qrs[:, j:j + 1]
        qn = qrs[:, j + 1:j + 2]
        if j == 0:
            left = jnp.zeros_like(qj)
        else:
            qp = qrs[:, j - 1:j]
            left = (qp + qj) // 2
        right = jnp.where(m == j + 1, Ld, (qj + qn) // 2)
        right = jnp.minimum(right, Ld)
        ws = jnp.maximum(right - left, 1)
        off = jnp.where(ws < W, (W - ws) // 2, -((ws - W) // 2))
        idx_j = left + jnp.clip(pp - off, 0, ws - 1)
        idx_cols.append(jnp.clip(idx_j, 0, Ld - 1))
    idx_all = jnp.concatenate(idx_cols, axis=1)

    jm = (jax.lax.broadcasted_iota(jnp.int32, (B, P * W), 1) // W) < m
    bb = jax.lax.broadcasted_iota(jnp.int32, (B, P * W), 0)
    bglob = bb + pl.program_id(0) * B
    idx_ref[...] = jnp.where(jm, idx_all, Ld) + _ROWPAD * bglob

    ps2 = jax.lax.broadcasted_iota(jnp.int32, (B, S), 1)
    vals = qrs // 100 + 1
    pmod = ps2 % m
    t = jnp.zeros((B, S), jnp.int32)
    for s in range(_MAXPEAKS):
        t = t + jnp.where(pmod == s, vals[:, s:s + 1], 0)
    act = ps2 < 12 * m
    t_ref[...] = jnp.where(act, t, 0)
    s_ref[...] = jnp.where(act, ps2 // m + 1, 0)


def _sc_gather_fn(nrows):
    nw = _NC * _NS
    per_w = nrows // nw
    chunks = per_w // 128
    assert per_w * nw == nrows and chunks * 128 == per_w

    mesh = plsc.VectorSubcoreMesh(core_axis_name="c", subcore_axis_name="s",
                                  num_cores=_NC, num_subcores=_NS)

    @functools.partial(
        pl.kernel,
        out_type=jax.ShapeDtypeStruct((nrows, _LANES), jnp.float32),
        mesh=mesh,
        scratch_types=[
            pltpu.VMEM((chunks, 128), jnp.int32),
            pltpu.VMEM((128, _LANES), jnp.float32),
            pltpu.VMEM((128, _LANES), jnp.float32),
            pltpu.SemaphoreType.DMA,
            pltpu.SemaphoreType.DMA,
        ],
    )
    def k(tbl_hbm, idx_hbm, out_hbm, idx_v, rows_a, rows_b, sem_a, sem_b):
        wid = lax.axis_index("s") * _NC + lax.axis_index("c")
        base = wid * per_w
        pltpu.sync_copy(idx_hbm.at[wid], idx_v)
        bufs = (rows_a, rows_b)
        sems = (sem_a, sem_b)
        cps = [None, None]
        cps[0] = pltpu.async_copy(tbl_hbm.at[idx_v.at[0]], bufs[0], sems[0])
        for h in range(1, chunks):
            cps[h % 2] = pltpu.async_copy(
                tbl_hbm.at[idx_v.at[h]], bufs[h % 2], sems[h % 2])
            cps[(h - 1) % 2].wait()
            pltpu.sync_copy(bufs[(h - 1) % 2],
                            out_hbm.at[pl.ds(base + (h - 1) * 128, 128)])
        cps[(chunks - 1) % 2].wait()
        pltpu.sync_copy(bufs[(chunks - 1) % 2],
                        out_hbm.at[pl.ds(base + (chunks - 1) * 128, 128)])

    return k


def kernel(x):
    B, C, L = x.shape
    Ld = _LD
    ph = jnp.reshape(x, (B, C, Ld, 5))[..., 1:4]
    xt0 = jnp.transpose(ph[:, 0], (0, 2, 1))
    xtt = jnp.transpose(ph, (0, 3, 2, 1))
    w = jnp.asarray(_W3)
    wt = jnp.asarray(_W3[:, :, None])

    nblk = B // _BBLK
    tbl, idxf, t, s = pl.pallas_call(
        _qrs_body,
        grid=(nblk,),
        in_specs=[
            pl.BlockSpec((_BBLK, 3, Ld), lambda i: (i, 0, 0)),
            pl.BlockSpec((_BBLK, 3, Ld, _NLEAD), lambda i: (i, 0, 0, 0)),
            pl.BlockSpec((3, Ld), lambda i: (0, 0)),
            pl.BlockSpec((3, Ld, 1), lambda i: (0, 0, 0)),
        ],
        out_specs=[
            pl.BlockSpec((_BBLK, _ROWPAD, _LANES), lambda i: (i, 0, 0)),
            pl.BlockSpec((_BBLK, _NPATCH * _WINDOW), lambda i: (i, 0)),
            pl.BlockSpec((_BBLK, _SENT), lambda i: (i, 0)),
            pl.BlockSpec((_BBLK, _SENT), lambda i: (i, 0)),
        ],
        out_shape=[
            jax.ShapeDtypeStruct((B, _ROWPAD, _LANES), jnp.float32),
            jax.ShapeDtypeStruct((B, _NPATCH * _WINDOW), jnp.int32),
            jax.ShapeDtypeStruct((B, _SENT), jnp.int32),
            jax.ShapeDtypeStruct((B, _SENT), jnp.int32),
        ],
    )(xt0, xtt, w, wt)

    nrows = B * _NPATCH * _WINDOW
    nw = _NC * _NS
    chunks = nrows // (128 * nw)
    idx3 = jnp.reshape(idxf, (nw, chunks, 128))
    rows = _sc_gather_fn(nrows)(
        jnp.reshape(tbl, (B * _ROWPAD, _LANES)), idx3)

    patches = jnp.transpose(
        jnp.reshape(rows[:, :_NLEAD], (B, _NPATCH, _WINDOW, _NLEAD)),
        (0, 1, 3, 2))
    x_pad = jnp.reshape(patches, (B, _SENT, _WINDOW))
    x_pad = jnp.reshape(x_pad, (B, 12, -1, _WINDOW))
    return (x_pad, t, s)


# speedup vs baseline: 1.0027x; 1.0027x over previous
import functools

import numpy as np
import jax
import jax.numpy as jnp
from jax import lax
from jax.experimental import pallas as pl
from jax.experimental.pallas import tpu as pltpu
from jax.experimental.pallas import tpu_sc as plsc

_WINDOW = 96
_SENT = 120
_FS = 500
_DOWN_FS = 100
_NPATCH = _SENT // 12
_MAXPEAKS = 25
_ROWPAD = 1024
_NLEAD = 12
_LANES = 128
_BBLK = 8
_NC = 2
_NS = 16


def _downsample_weights(L=5000, scale=_DOWN_FS / _FS):
    Lo = int(L * scale)
    src = (np.arange(Lo) + 0.5) / scale - 0.5
    src = np.clip(src, 0.0, L - 1)
    lo = np.floor(src).astype(np.int64)
    hi = np.minimum(lo + 1, L - 1)
    frac = (src - lo).astype(np.float32)
    base = 5 * np.arange(Lo)
    olo = lo - base - 1
    ohi = hi - base - 1
    assert olo.min() >= 0 and ohi.max() <= 2, "static interp offsets changed"
    w = np.zeros((3, Lo), np.float32)
    w[olo, np.arange(Lo)] += (1.0 - frac)
    w[ohi, np.arange(Lo)] += frac
    return w


_W3 = _downsample_weights()
_LD = _W3.shape[1]


def _qrs_body(xt0_ref, xtt_ref, w_ref, wt_ref,
              tbl_ref, idx_ref, t_ref, s_ref):
    B = xt0_ref.shape[0]
    Ld = _LD
    W = _WINDOW
    P = _NPATCH
    S = _SENT

    xt0 = xt0_ref[...]
    w = w_ref[...]
    sig = (xt0[:, 0] * w[0:1, :] + xt0[:, 1] * w[1:2, :]
           + xt0[:, 2] * w[2:3, :])

    xtt = xtt_ref[...]
    wt = wt_ref[...]
    xds_t = (xtt[:, 0] * wt[0] + xtt[:, 1] * wt[1]
             + xtt[:, 2] * wt[2])
    tbl_ref[...] = jnp.zeros((B * _ROWPAD, _LANES), jnp.float32)
    for b in range(B):
        tbl_ref[b * _ROWPAD:b * _ROWPAD + Ld, 0:_NLEAD] = xds_t[b]

    mu = jnp.mean(sig, axis=1, keepdims=True)
    var = jnp.mean((sig - mu) ** 2, axis=1, keepdims=True)
    thr = mu + 1.5 * jnp.sqrt(var)
    sl = jnp.concatenate([sig[:, :1], sig[:, :-1]], axis=1)
    sr = jnp.concatenate([sig[:, 1:], sig[:, -1:]], axis=1)
    ii = jax.lax.broadcasted_iota(jnp.int32, (B, Ld), 1)
    cand = ((sig > sl) & (sig >= sr) & (sig > thr)
            & (ii >= 1) & (ii <= Ld - 2))

    amp = jnp.where(cand, sig, -jnp.inf)
    kept = jnp.zeros((B, Ld), dtype=jnp.bool_)
    for _ in range(_MAXPEAKS):
        rowmax = jnp.max(amp, axis=1, keepdims=True)
        valid = rowmax > -jnp.inf
        pm = jnp.min(jnp.where(amp == rowmax, ii, Ld), axis=1, keepdims=True)
        kept = kept | (valid & (ii == pm))
        amp = jnp.where(valid & (jnp.abs(ii - pm) < 40), -jnp.inf, amp)

    has = jnp.any(kept, axis=1, keepdims=True)
    smax = jnp.max(sig, axis=1, keepdims=True)
    pmax = jnp.min(jnp.where(sig == smax, ii, Ld), axis=1, keepdims=True)
    kept = (kept & has) | ((ii == pmax) & jnp.logical_not(has))
    m = jnp.sum(kept.astype(jnp.int32), axis=1, keepdims=True)

    cols = []
    kk = kept
    for _ in range(_MAXPEAKS):
        ps = jnp.min(jnp.where(kk, ii, Ld), axis=1, keepdims=True)
        cols.append(jnp.where(ps < Ld, ps, 0))
        kk = kk & (ii != ps)
    qrs = jnp.concatenate(cols, axis=1)

    pp = jax.lax.broadcasted_iota(jnp.int32, (B, W), 1)
    idx_cols = []
    for j in range(P):
        qj = qrs[:, j:j + 1]
        qn = qrs[:, j + 1:j + 2]
        if j == 0:
            left = jnp.zeros_like(qj)
        else:
            qp = qrs[:, j - 1:j]
            left = (qp + qj) // 2
        right = jnp.where(m == j + 1, Ld, (qj + qn) // 2)
        right = jnp.minimum(right, Ld)
        ws = jnp.maximum(right - left, 1)
        off = jnp.where(ws < W, (W - ws) // 2, -((ws - W) // 2))
        idx_j = left + jnp.clip(pp - off, 0, ws - 1)
        idx_cols.append(jnp.clip(idx_j, 0, Ld - 1))
    idx_all = jnp.concatenate(idx_cols, axis=1)

    jm = (jax.lax.broadcasted_iota(jnp.int32, (B, P * W), 1) // W) < m
    bb = jax.lax.broadcasted_iota(jnp.int32, (B, P * W), 0)
    bglob = bb + pl.program_id(0) * B
    idx_ref[...] = jnp.where(jm, idx_all, Ld) + _ROWPAD * bglob

    ps2 = jax.lax.broadcasted_iota(jnp.int32, (B, S), 1)
    vals = qrs // 100 + 1
    pmod = ps2 % m
    t = jnp.zeros((B, S), jnp.int32)
    for s in range(_MAXPEAKS):
        t = t + jnp.where(pmod == s, vals[:, s:s + 1], 0)
    act = ps2 < 12 * m
    t_ref[...] = jnp.where(act, t, 0)
    s_ref[...] = jnp.where(act, ps2 // m + 1, 0)


def _sc_gather_fn(nrows):
    nw = _NC * _NS
    per_w = nrows // nw
    chunks = per_w // 128
    assert per_w * nw == nrows and chunks * 128 == per_w

    mesh = plsc.VectorSubcoreMesh(core_axis_name="c", subcore_axis_name="s",
                                  num_cores=_NC, num_subcores=_NS)

    @functools.partial(
        pl.kernel,
        out_type=jax.ShapeDtypeStruct((nrows, _LANES), jnp.float32),
        mesh=mesh,
        scratch_types=[
            pltpu.VMEM((chunks, 128), jnp.int32),
            pltpu.VMEM((128, _LANES), jnp.float32),
            pltpu.VMEM((128, _LANES), jnp.float32),
            pltpu.SemaphoreType.DMA,
            pltpu.SemaphoreType.DMA,
        ],
    )
    def k(tbl_hbm, idx_hbm, out_hbm, idx_v, rows_a, rows_b, sem_a, sem_b):
        wid = lax.axis_index("s") * _NC + lax.axis_index("c")
        base = wid * per_w
        pltpu.sync_copy(idx_hbm.at[wid], idx_v)
        bufs = (rows_a, rows_b)
        sems = (sem_a, sem_b)
        cps = [None, None]
        cps[0] = pltpu.async_copy(tbl_hbm.at[idx_v.at[0]], bufs[0], sems[0])
        for h in range(1, chunks):
            cps[h % 2] = pltpu.async_copy(
                tbl_hbm.at[idx_v.at[h]], bufs[h % 2], sems[h % 2])
            cps[(h - 1) % 2].wait()
            pltpu.sync_copy(bufs[(h - 1) % 2],
                            out_hbm.at[pl.ds(base + (h - 1) * 128, 128)])
        cps[(chunks - 1) % 2].wait()
        pltpu.sync_copy(bufs[(chunks - 1) % 2],
                        out_hbm.at[pl.ds(base + (chunks - 1) * 128, 128)])

    return k


def kernel(x):
    B, C, L = x.shape
    Ld = _LD
    ph = jnp.reshape(x, (B, C, Ld, 5))[..., 1:4]
    xt0 = jnp.transpose(ph[:, 0], (0, 2, 1))
    xtt = jnp.transpose(ph, (0, 3, 2, 1))
    w = jnp.asarray(_W3)
    wt = jnp.asarray(_W3[:, :, None])

    nblk = B // _BBLK
    tbl, idxf, t, s = pl.pallas_call(
        _qrs_body,
        grid=(nblk,),
        in_specs=[
            pl.BlockSpec((_BBLK, 3, Ld), lambda i: (i, 0, 0)),
            pl.BlockSpec((_BBLK, 3, Ld, _NLEAD), lambda i: (i, 0, 0, 0)),
            pl.BlockSpec((3, Ld), lambda i: (0, 0)),
            pl.BlockSpec((3, Ld, 1), lambda i: (0, 0, 0)),
        ],
        out_specs=[
            pl.BlockSpec((_BBLK * _ROWPAD, _LANES), lambda i: (i, 0)),
            pl.BlockSpec((_BBLK, _NPATCH * _WINDOW), lambda i: (i, 0)),
            pl.BlockSpec((_BBLK, _SENT), lambda i: (i, 0)),
            pl.BlockSpec((_BBLK, _SENT), lambda i: (i, 0)),
        ],
        out_shape=[
            jax.ShapeDtypeStruct((B * _ROWPAD, _LANES), jnp.float32),
            jax.ShapeDtypeStruct((B, _NPATCH * _WINDOW), jnp.int32),
            jax.ShapeDtypeStruct((B, _SENT), jnp.int32),
            jax.ShapeDtypeStruct((B, _SENT), jnp.int32),
        ],
    )(xt0, xtt, w, wt)

    nrows = B * _NPATCH * _WINDOW
    nw = _NC * _NS
    chunks = nrows // (128 * nw)
    idx3 = jnp.reshape(idxf, (nw, chunks, 128))
    rows = _sc_gather_fn(nrows)(tbl, idx3)

    patches = jnp.transpose(
        jnp.reshape(rows[:, :_NLEAD], (B, _NPATCH, _WINDOW, _NLEAD)),
        (0, 1, 3, 2))
    x_pad = jnp.reshape(patches, (B, _SENT, _WINDOW))
    x_pad = jnp.reshape(x_pad, (B, 12, -1, _WINDOW))
    return (x_pad, t, s)


# speedup vs baseline: 1.0506x; 1.0478x over previous
import functools

import numpy as np
import jax
import jax.numpy as jnp
from jax import lax
from jax.experimental import pallas as pl
from jax.experimental.pallas import tpu as pltpu
from jax.experimental.pallas import tpu_sc as plsc

_WINDOW = 96
_SENT = 120
_FS = 500
_DOWN_FS = 100
_NPATCH = _SENT // 12
_MAXPEAKS = 25
_NLEAD = 12
_LANES = 128
_NBLKS = 9
_BBLK = 8
_NC = 2
_NS = 16


def _downsample_weights(L=5000, scale=_DOWN_FS / _FS):
    Lo = int(L * scale)
    src = (np.arange(Lo) + 0.5) / scale - 0.5
    src = np.clip(src, 0.0, L - 1)
    lo = np.floor(src).astype(np.int64)
    hi = np.minimum(lo + 1, L - 1)
    frac = (src - lo).astype(np.float32)
    base = 5 * np.arange(Lo)
    olo = lo - base - 1
    ohi = hi - base - 1
    assert olo.min() >= 0 and ohi.max() <= 2, "static interp offsets changed"
    w = np.zeros((3, Lo), np.float32)
    w[olo, np.arange(Lo)] += (1.0 - frac)
    w[ohi, np.arange(Lo)] += frac
    return w


_W3 = _downsample_weights()
_LD = _W3.shape[1]


def _qrs_body(xtt_ref, w_ref, tbl_ref, grow_ref, idxr_ref, t_ref, s_ref):
    B = xtt_ref.shape[0]
    Ld = _LD
    W = _WINDOW
    P = _NPATCH
    S = _SENT

    xtt = xtt_ref[...]
    w = w_ref[...]
    ds = (xtt[:, :, 0] * w[0][None, None, :]
          + xtt[:, :, 1] * w[1][None, None, :]
          + xtt[:, :, 2] * w[2][None, None, :])
    sig = ds[:, 0]

    dsp = jnp.concatenate(
        [ds, jnp.zeros((B, _NLEAD, _NBLKS * _LANES - Ld), jnp.float32)],
        axis=2)
    for k in range(_NBLKS):
        tbl_ref[:, k, :] = jnp.reshape(
            dsp[:, :, k * _LANES:(k + 1) * _LANES], (B * _NLEAD, _LANES))

    mu = jnp.mean(sig, axis=1, keepdims=True)
    var = jnp.mean((sig - mu) ** 2, axis=1, keepdims=True)
    thr = mu + 1.5 * jnp.sqrt(var)
    sl = jnp.concatenate([sig[:, :1], sig[:, :-1]], axis=1)
    sr = jnp.concatenate([sig[:, 1:], sig[:, -1:]], axis=1)
    ii = jax.lax.broadcasted_iota(jnp.int32, (B, Ld), 1)
    cand = ((sig > sl) & (sig >= sr) & (sig > thr)
            & (ii >= 1) & (ii <= Ld - 2))

    amp = jnp.where(cand, sig, -jnp.inf)
    kept = jnp.zeros((B, Ld), dtype=jnp.bool_)
    for _ in range(_MAXPEAKS):
        rowmax = jnp.max(amp, axis=1, keepdims=True)
        valid = rowmax > -jnp.inf
        pm = jnp.min(jnp.where(amp == rowmax, ii, Ld), axis=1, keepdims=True)
        kept = kept | (valid & (ii == pm))
        amp = jnp.where(valid & (jnp.abs(ii - pm) < 40), -jnp.inf, amp)

    has = jnp.any(kept, axis=1, keepdims=True)
    smax = jnp.max(sig, axis=1, keepdims=True)
    pmax = jnp.min(jnp.where(sig == smax, ii, Ld), axis=1, keepdims=True)
    kept = (kept & has) | ((ii == pmax) & jnp.logical_not(has))
    m = jnp.sum(kept.astype(jnp.int32), axis=1, keepdims=True)

    cols = []
    kk = kept
    for _ in range(_MAXPEAKS):
        ps = jnp.min(jnp.where(kk, ii, Ld), axis=1, keepdims=True)
        cols.append(jnp.where(ps < Ld, ps, 0))
        kk = kk & (ii != ps)
    qrs = jnp.concatenate(cols, axis=1)

    pp = jax.lax.broadcasted_iota(jnp.int32, (B, W), 1)
    blks = []
    for j in range(P):
        qj = qrs[:, j:j + 1]
        qn = qrs[:, j + 1:j + 2]
        if j == 0:
            left = jnp.zeros_like(qj)
        else:
            qp = qrs[:, j - 1:j]
            left = (qp + qj) // 2
        right = jnp.where(m == j + 1, Ld, (qj + qn) // 2)
        right = jnp.minimum(right, Ld)
        ws = jnp.maximum(right - left, 1)
        off = jnp.where(ws < W, (W - ws) // 2, -((ws - W) // 2))
        idx_j = left + jnp.clip(pp - off, 0, ws - 1)
        idx_j = jnp.clip(idx_j, 0, Ld - 1)
        blk_j = idx_j[:, 0:1] // _LANES
        blks.append(blk_j)
        idxr_ref[:, j, :] = jnp.where(m > j, idx_j - _LANES * blk_j, 300)
    blk = jnp.concatenate(blks, axis=1)

    bglob = (jax.lax.broadcasted_iota(jnp.int32, (B, P, _NLEAD, 2), 0)
             + pl.program_id(0) * B)
    li = jax.lax.broadcasted_iota(jnp.int32, (B, P, _NLEAD, 2), 2)
    hi2 = jax.lax.broadcasted_iota(jnp.int32, (B, P, _NLEAD, 2), 3)
    grow_ref[...] = ((bglob * _NLEAD + li) * _NBLKS
                     + blk[:, :, None, None] + hi2)

    ps2 = jax.lax.broadcasted_iota(jnp.int32, (B, S), 1)
    vals = qrs // 100 + 1
    pmod = ps2 % m
    t = jnp.zeros((B, S), jnp.int32)
    for s in range(_MAXPEAKS):
        t = t + jnp.where(pmod == s, vals[:, s:s + 1], 0)
    act = ps2 < 12 * m
    t_ref[...] = jnp.where(act, t, 0)
    s_ref[...] = jnp.where(act, ps2 // m + 1, 0)


def _unpack_body(rows_ref, idxr_ref, out_ref):
    B = idxr_ref.shape[0]
    P = _NPATCH
    W = _WINDOW
    G = B * P
    rows = jnp.reshape(rows_ref[...], (G, _NLEAD, 2, _LANES))
    ir = jnp.reshape(idxr_ref[...], (G, W))
    qq = jax.lax.broadcasted_iota(jnp.int32, (G, W, _LANES), 2)
    dn = (((2,), (2,)), ((0,), (0,)))
    out = jnp.zeros((G, _NLEAD, W), jnp.float32)
    for h in range(2):
        sel_h = (ir[:, :, None] == qq + h * _LANES).astype(jnp.float32)
        out = out + lax.dot_general(
            rows[:, :, h, :], sel_h, dimension_numbers=dn,
            preferred_element_type=jnp.float32)
    out_ref[...] = jnp.reshape(out, (B, P, _NLEAD, W))


def _sc_gather_fn(nrows):
    nw = _NC * _NS
    per_w = nrows // nw
    chunks = per_w // 128
    assert per_w * nw == nrows and chunks * 128 == per_w

    mesh = plsc.VectorSubcoreMesh(core_axis_name="c", subcore_axis_name="s",
                                  num_cores=_NC, num_subcores=_NS)

    @functools.partial(
        pl.kernel,
        out_type=jax.ShapeDtypeStruct((nrows, _LANES), jnp.float32),
        mesh=mesh,
        scratch_types=[
            pltpu.VMEM((chunks, 128), jnp.int32),
            pltpu.VMEM((128, _LANES), jnp.float32),
            pltpu.VMEM((128, _LANES), jnp.float32),
            pltpu.SemaphoreType.DMA,
            pltpu.SemaphoreType.DMA,
        ],
    )
    def k(tbl_hbm, idx_hbm, out_hbm, idx_v, rows_a, rows_b, sem_a, sem_b):
        wid = lax.axis_index("s") * _NC + lax.axis_index("c")
        base = wid * per_w
        pltpu.sync_copy(idx_hbm.at[wid], idx_v)
        bufs = (rows_a, rows_b)
        sems = (sem_a, sem_b)
        cps = [None, None]
        cps[0] = pltpu.async_copy(tbl_hbm.at[idx_v.at[0]], bufs[0], sems[0])
        for h in range(1, chunks):
            cps[h % 2] = pltpu.async_copy(
                tbl_hbm.at[idx_v.at[h]], bufs[h % 2], sems[h % 2])
            cps[(h - 1) % 2].wait()
            pltpu.sync_copy(bufs[(h - 1) % 2],
                            out_hbm.at[pl.ds(base + (h - 1) * 128, 128)])
        cps[(chunks - 1) % 2].wait()
        pltpu.sync_copy(bufs[(chunks - 1) % 2],
                        out_hbm.at[pl.ds(base + (chunks - 1) * 128, 128)])

    return k


def kernel(x):
    B, C, L = x.shape
    Ld = _LD
    ph = jnp.reshape(x, (B, C, Ld, 5))[..., 1:4]
    xtt = jnp.transpose(ph, (0, 1, 3, 2))
    w = jnp.asarray(_W3)

    nblk = B // _BBLK
    tbl, grow, idxr, t, s = pl.pallas_call(
        _qrs_body,
        grid=(nblk,),
        in_specs=[
            pl.BlockSpec((_BBLK, _NLEAD, 3, Ld), lambda i: (i, 0, 0, 0)),
            pl.BlockSpec((3, Ld), lambda i: (0, 0)),
        ],
        out_specs=[
            pl.BlockSpec((_BBLK * _NLEAD, _NBLKS, _LANES),
                         lambda i: (i, 0, 0)),
            pl.BlockSpec((_BBLK, _NPATCH, _NLEAD, 2), lambda i: (i, 0, 0, 0)),
            pl.BlockSpec((_BBLK, _NPATCH, _WINDOW), lambda i: (i, 0, 0)),
            pl.BlockSpec((_BBLK, _SENT), lambda i: (i, 0)),
            pl.BlockSpec((_BBLK, _SENT), lambda i: (i, 0)),
        ],
        out_shape=[
            jax.ShapeDtypeStruct((B * _NLEAD, _NBLKS, _LANES), jnp.float32),
            jax.ShapeDtypeStruct((B, _NPATCH, _NLEAD, 2), jnp.int32),
            jax.ShapeDtypeStruct((B, _NPATCH, _WINDOW), jnp.int32),
            jax.ShapeDtypeStruct((B, _SENT), jnp.int32),
            jax.ShapeDtypeStruct((B, _SENT), jnp.int32),
        ],
    )(xtt, w)

    nreal = B * _NPATCH * _NLEAD * 2
    nw = _NC * _NS
    nrows = ((nreal + 128 * nw - 1) // (128 * nw)) * 128 * nw
    gflat = jnp.concatenate(
        [jnp.reshape(grow, (nreal,)),
         jnp.zeros((nrows - nreal,), jnp.int32)])
    idx3 = jnp.reshape(gflat, (nw, nrows // (128 * nw), 128))
    rows = _sc_gather_fn(nrows)(
        jnp.reshape(tbl, (B * _NLEAD * _NBLKS, _LANES)), idx3)

    per_step = _BBLK * _NPATCH * _NLEAD * 2
    patches = pl.pallas_call(
        _unpack_body,
        grid=(nblk,),
        in_specs=[
            pl.BlockSpec((per_step, _LANES), lambda i: (i, 0)),
            pl.BlockSpec((_BBLK, _NPATCH, _WINDOW), lambda i: (i, 0, 0)),
        ],
        out_specs=pl.BlockSpec((_BBLK, _NPATCH, _NLEAD, _WINDOW),
                               lambda i: (i, 0, 0, 0)),
        out_shape=jax.ShapeDtypeStruct((B, _NPATCH, _NLEAD, _WINDOW),
                                       jnp.float32),
    )(rows, idxr)

    x_pad = jnp.reshape(patches, (B, 12, -1, _WINDOW))
    return (x_pad, t, s)


# speedup vs baseline: 1.4384x; 1.3691x over previous
import functools

import numpy as np
import jax
import jax.numpy as jnp
from jax import lax
from jax.experimental import pallas as pl
from jax.experimental.pallas import tpu as pltpu
from jax.experimental.pallas import tpu_sc as plsc

_WINDOW = 96
_SENT = 120
_FS = 500
_DOWN_FS = 100
_NPATCH = _SENT // 12
_MAXPEAKS = 25
_NLEAD = 12
_LANES = 128
_NBLKS = 9
_BBLK = 64
_UBLK = 8
_NC = 2
_NS = 16


def _downsample_weights(L=5000, scale=_DOWN_FS / _FS):
    Lo = int(L * scale)
    src = (np.arange(Lo) + 0.5) / scale - 0.5
    src = np.clip(src, 0.0, L - 1)
    lo = np.floor(src).astype(np.int64)
    hi = np.minimum(lo + 1, L - 1)
    frac = (src - lo).astype(np.float32)
    base = 5 * np.arange(Lo)
    olo = lo - base - 1
    ohi = hi - base - 1
    assert olo.min() >= 0 and ohi.max() <= 2, "static interp offsets changed"
    w = np.zeros((3, Lo), np.float32)
    w[olo, np.arange(Lo)] += (1.0 - frac)
    w[ohi, np.arange(Lo)] += frac
    return w


_W3 = _downsample_weights()
_LD = _W3.shape[1]


def _qrs_body(xtt_ref, w_ref, tbl_ref, grow_ref, idxr_ref, t_ref, s_ref):
    B = xtt_ref.shape[0]
    Ld = _LD
    W = _WINDOW
    P = _NPATCH
    S = _SENT

    xtt = xtt_ref[...]
    w = w_ref[...]
    ds = (xtt[:, :, 0] * w[0][None, None, :]
          + xtt[:, :, 1] * w[1][None, None, :]
          + xtt[:, :, 2] * w[2][None, None, :])
    sig = ds[:, 0]

    dsp = jnp.concatenate(
        [ds, jnp.zeros((B, _NLEAD, _NBLKS * _LANES - Ld), jnp.float32)],
        axis=2)
    for k in range(_NBLKS):
        tbl_ref[:, k, :] = jnp.reshape(
            dsp[:, :, k * _LANES:(k + 1) * _LANES], (B * _NLEAD, _LANES))

    mu = jnp.mean(sig, axis=1, keepdims=True)
    var = jnp.mean((sig - mu) ** 2, axis=1, keepdims=True)
    thr = mu + 1.5 * jnp.sqrt(var)
    sl = jnp.concatenate([sig[:, :1], sig[:, :-1]], axis=1)
    sr = jnp.concatenate([sig[:, 1:], sig[:, -1:]], axis=1)
    ii = jax.lax.broadcasted_iota(jnp.int32, (B, Ld), 1)
    cand = ((sig > sl) & (sig >= sr) & (sig > thr)
            & (ii >= 1) & (ii <= Ld - 2))

    amp = jnp.where(cand, sig, -jnp.inf)
    kept = jnp.zeros((B, Ld), dtype=jnp.bool_)
    for _ in range(_MAXPEAKS):
        rowmax = jnp.max(amp, axis=1, keepdims=True)
        valid = rowmax > -jnp.inf
        pm = jnp.min(jnp.where(amp == rowmax, ii, Ld), axis=1, keepdims=True)
        kept = kept | (valid & (ii == pm))
        amp = jnp.where(valid & (jnp.abs(ii - pm) < 40), -jnp.inf, amp)

    has = jnp.any(kept, axis=1, keepdims=True)
    smax = jnp.max(sig, axis=1, keepdims=True)
    pmax = jnp.min(jnp.where(sig == smax, ii, Ld), axis=1, keepdims=True)
    kept = (kept & has) | ((ii == pmax) & jnp.logical_not(has))
    m = jnp.sum(kept.astype(jnp.int32), axis=1, keepdims=True)

    cols = []
    kk = kept
    for _ in range(_MAXPEAKS):
        ps = jnp.min(jnp.where(kk, ii, Ld), axis=1, keepdims=True)
        cols.append(jnp.where(ps < Ld, ps, 0))
        kk = kk & (ii != ps)
    qrs = jnp.concatenate(cols, axis=1)

    pp = jax.lax.broadcasted_iota(jnp.int32, (B, W), 1)
    blks = []
    for j in range(P):
        qj = qrs[:, j:j + 1]
        qn = qrs[:, j + 1:j + 2]
        if j == 0:
            left = jnp.zeros_like(qj)
        else:
            qp = qrs[:, j - 1:j]
            left = (qp + qj) // 2
        right = jnp.where(m == j + 1, Ld, (qj + qn) // 2)
        right = jnp.minimum(right, Ld)
        ws = jnp.maximum(right - left, 1)
        off = jnp.where(ws < W, (W - ws) // 2, -((ws - W) // 2))
        idx_j = left + jnp.clip(pp - off, 0, ws - 1)
        idx_j = jnp.clip(idx_j, 0, Ld - 1)
        blk_j = idx_j[:, 0:1] // _LANES
        blks.append(blk_j)
        idxr_ref[:, j, :] = jnp.where(m > j, idx_j - _LANES * blk_j, 300)
    blk = jnp.concatenate(blks, axis=1)

    bglob = (jax.lax.broadcasted_iota(jnp.int32, (B, P, _NLEAD, 2), 0)
             + pl.program_id(0) * B)
    li = jax.lax.broadcasted_iota(jnp.int32, (B, P, _NLEAD, 2), 2)
    hi2 = jax.lax.broadcasted_iota(jnp.int32, (B, P, _NLEAD, 2), 3)
    grow_ref[...] = ((bglob * _NLEAD + li) * _NBLKS
                     + blk[:, :, None, None] + hi2)

    ps2 = jax.lax.broadcasted_iota(jnp.int32, (B, S), 1)
    vals = qrs // 100 + 1
    pmod = ps2 % m
    t = jnp.zeros((B, S), jnp.int32)
    for s in range(_MAXPEAKS):
        t = t + jnp.where(pmod == s, vals[:, s:s + 1], 0)
    act = ps2 < 12 * m
    t_ref[...] = jnp.where(act, t, 0)
    s_ref[...] = jnp.where(act, ps2 // m + 1, 0)


def _unpack_body(rows_ref, idxr_ref, out_ref):
    B = idxr_ref.shape[0]
    P = _NPATCH
    W = _WINDOW
    G = B * P
    rows = jnp.reshape(rows_ref[...], (G, _NLEAD, 2, _LANES))
    ir = jnp.reshape(idxr_ref[...], (G, W))
    qq = jax.lax.broadcasted_iota(jnp.int32, (G, W, _LANES), 2)
    dn = (((2,), (2,)), ((0,), (0,)))
    out = jnp.zeros((G, _NLEAD, W), jnp.float32)
    for h in range(2):
        sel_h = (ir[:, :, None] == qq + h * _LANES).astype(jnp.float32)
        out = out + lax.dot_general(
            rows[:, :, h, :], sel_h, dimension_numbers=dn,
            preferred_element_type=jnp.float32)
    out_ref[...] = jnp.reshape(out, (B, P, _NLEAD, W))


def _sc_gather_fn(nrows):
    nw = _NC * _NS
    per_w = nrows // nw
    chunks = per_w // 128
    assert per_w * nw == nrows and chunks * 128 == per_w

    mesh = plsc.VectorSubcoreMesh(core_axis_name="c", subcore_axis_name="s",
                                  num_cores=_NC, num_subcores=_NS)

    @functools.partial(
        pl.kernel,
        out_type=jax.ShapeDtypeStruct((nrows, _LANES), jnp.float32),
        mesh=mesh,
        scratch_types=[
            pltpu.VMEM((chunks, 128), jnp.int32),
            pltpu.VMEM((128, _LANES), jnp.float32),
            pltpu.VMEM((128, _LANES), jnp.float32),
            pltpu.SemaphoreType.DMA,
            pltpu.SemaphoreType.DMA,
        ],
    )
    def k(tbl_hbm, idx_hbm, out_hbm, idx_v, rows_a, rows_b, sem_a, sem_b):
        wid = lax.axis_index("s") * _NC + lax.axis_index("c")
        base = wid * per_w
        pltpu.sync_copy(idx_hbm.at[wid], idx_v)
        bufs = (rows_a, rows_b)
        sems = (sem_a, sem_b)
        cps = [None, None]
        cps[0] = pltpu.async_copy(tbl_hbm.at[idx_v.at[0]], bufs[0], sems[0])
        for h in range(1, chunks):
            cps[h % 2] = pltpu.async_copy(
                tbl_hbm.at[idx_v.at[h]], bufs[h % 2], sems[h % 2])
            cps[(h - 1) % 2].wait()
            pltpu.sync_copy(bufs[(h - 1) % 2],
                            out_hbm.at[pl.ds(base + (h - 1) * 128, 128)])
        cps[(chunks - 1) % 2].wait()
        pltpu.sync_copy(bufs[(chunks - 1) % 2],
                        out_hbm.at[pl.ds(base + (chunks - 1) * 128, 128)])

    return k


def kernel(x):
    B, C, L = x.shape
    Ld = _LD
    ph = jnp.reshape(x, (B, C, Ld, 5))[..., 1:4]
    xtt = jnp.transpose(ph, (0, 1, 3, 2))
    w = jnp.asarray(_W3)

    nblk = B // _BBLK
    tbl, grow, idxr, t, s = pl.pallas_call(
        _qrs_body,
        grid=(nblk,),
        in_specs=[
            pl.BlockSpec((_BBLK, _NLEAD, 3, Ld), lambda i: (i, 0, 0, 0)),
            pl.BlockSpec((3, Ld), lambda i: (0, 0)),
        ],
        out_specs=[
            pl.BlockSpec((_BBLK * _NLEAD, _NBLKS, _LANES),
                         lambda i: (i, 0, 0)),
            pl.BlockSpec((_BBLK, _NPATCH, _NLEAD, 2), lambda i: (i, 0, 0, 0)),
            pl.BlockSpec((_BBLK, _NPATCH, _WINDOW), lambda i: (i, 0, 0)),
            pl.BlockSpec((_BBLK, _SENT), lambda i: (i, 0)),
            pl.BlockSpec((_BBLK, _SENT), lambda i: (i, 0)),
        ],
        out_shape=[
            jax.ShapeDtypeStruct((B * _NLEAD, _NBLKS, _LANES), jnp.float32),
            jax.ShapeDtypeStruct((B, _NPATCH, _NLEAD, 2), jnp.int32),
            jax.ShapeDtypeStruct((B, _NPATCH, _WINDOW), jnp.int32),
            jax.ShapeDtypeStruct((B, _SENT), jnp.int32),
            jax.ShapeDtypeStruct((B, _SENT), jnp.int32),
        ],
    )(xtt, w)

    nreal = B * _NPATCH * _NLEAD * 2
    nw = _NC * _NS
    nrows = ((nreal + 128 * nw - 1) // (128 * nw)) * 128 * nw
    gflat = jnp.concatenate(
        [jnp.reshape(grow, (nreal,)),
         jnp.zeros((nrows - nreal,), jnp.int32)])
    idx3 = jnp.reshape(gflat, (nw, nrows // (128 * nw), 128))
    rows = _sc_gather_fn(nrows)(
        jnp.reshape(tbl, (B * _NLEAD * _NBLKS, _LANES)), idx3)

    per_step = _UBLK * _NPATCH * _NLEAD * 2
    patches = pl.pallas_call(
        _unpack_body,
        grid=(B // _UBLK,),
        in_specs=[
            pl.BlockSpec((per_step, _LANES), lambda i: (i, 0)),
            pl.BlockSpec((_UBLK, _NPATCH, _WINDOW), lambda i: (i, 0, 0)),
        ],
        out_specs=pl.BlockSpec((_UBLK, _NPATCH, _NLEAD, _WINDOW),
                               lambda i: (i, 0, 0, 0)),
        out_shape=jax.ShapeDtypeStruct((B, _NPATCH, _NLEAD, _WINDOW),
                                       jnp.float32),
    )(rows, idxr)

    x_pad = jnp.reshape(patches, (B, 12, -1, _WINDOW))
    return (x_pad, t, s)
